# Initial kernel scaffold; baseline (speedup 1.0000x reference)
import jax
import jax.numpy as jnp
from jax.experimental import pallas as pl


def kernel(x, edge_index, Wc0, bc0, Wl0, bl0, Wc1, bc1, Wl1, bl1):
    raise NotImplementedError("write your pallas kernel here")


# speedup vs baseline: 5.7496x; 5.7496x over previous
import dataclasses
import functools

import jax
import jax.numpy as jnp
from jax import lax
from jax.experimental import pallas as pl
from jax.experimental.pallas import tpu as pltpu
from jax.experimental.pallas import tpu_sc as plsc

N = 10000
E = 320000
D = 128
H = 128
NPAD = 10240
NCORES = 2
NSUB = 16
NW = NCORES * NSUB
CHUNK = 128
NCHUNKS = E // CHUNK
ROWS_PER_SUB = NPAD // NSUB

_SC_MESH = plsc.VectorSubcoreMesh(core_axis_name="c", subcore_axis_name="s")

_CP = pltpu.CompilerParams()
if "needs_layout_passes" in pltpu.CompilerParams.__dataclass_fields__:
    _CP = dataclasses.replace(_CP, needs_layout_passes=False)


def _make_seg_sum():
    out_type = [jax.ShapeDtypeStruct((NCORES * NPAD, H), jnp.float32)]
    scratch = [
        pltpu.VMEM((CHUNK,), jnp.int32),
        pltpu.VMEM((CHUNK,), jnp.int32),
        pltpu.VMEM((CHUNK, H), jnp.float32),
        pltpu.VMEM_SHARED((NPAD, H), jnp.float32),
        pltpu.SemaphoreType.DMA,
    ]

    def body(hc_hbm, src_hbm, dst_hbm, acc_out, idx_s, idx_d, rows,
             acc_sh, sem):
        cid = lax.axis_index("c")
        sid = lax.axis_index("s")
        wid = cid * NSUB + sid
        base = sid * ROWS_PER_SUB
        zv = jnp.zeros((16,), jnp.float32)

        @pl.loop(0, CHUNK)
        def _(r):
            @pl.loop(0, H // 16)
            def _(c):
                rows[r, pl.ds(c * 16, 16)] = zv

        @pl.loop(0, ROWS_PER_SUB // CHUNK)
        def _(k):
            off = pl.multiple_of(base + k * CHUNK, CHUNK)
            pltpu.sync_copy(rows, acc_sh.at[pl.ds(off, CHUNK)])

        plsc.subcore_barrier()

        max_iters = (NCHUNKS + NW - 1) // NW

        @pl.loop(0, max_iters)
        def _(i):
            c = wid + i * NW

            @pl.when(c < NCHUNKS)
            def _():
                eoff = pl.multiple_of(c * CHUNK, CHUNK)
                pltpu.sync_copy(src_hbm.at[pl.ds(eoff, CHUNK)], idx_s)
                pltpu.sync_copy(dst_hbm.at[pl.ds(eoff, CHUNK)], idx_d)
                pltpu.async_copy(hc_hbm.at[idx_s], rows, sem).wait()
                pltpu.sync_copy(rows, acc_sh.at[idx_d], add=True)

        plsc.subcore_barrier()

        ooff = pl.multiple_of(cid * NPAD + base, CHUNK)
        pltpu.sync_copy(acc_sh.at[pl.ds(base, ROWS_PER_SUB)],
                        acc_out.at[pl.ds(ooff, ROWS_PER_SUB)])

    return pl.kernel(body, out_type=out_type, mesh=_SC_MESH,
                     scratch_types=scratch)


def _make_cnt():
    def body(dst_hbm, cnt_out, idx_d, cnt_v):
        cid = lax.axis_index("c")
        sid = lax.axis_index("s")
        wid = cid * NSUB + sid
        zv = jnp.zeros((16,), jnp.float32)
        ones16 = jnp.ones((16,), jnp.float32)

        @pl.loop(0, NPAD // 16)
        def _(r):
            cnt_v[pl.ds(r * 16, 16)] = zv

        max_iters = (NCHUNKS + NW - 1) // NW

        @pl.loop(0, max_iters)
        def _(i):
            c = wid + i * NW

            @pl.when(c < NCHUNKS)
            def _():
                eoff = pl.multiple_of(c * CHUNK, CHUNK)
                pltpu.sync_copy(dst_hbm.at[pl.ds(eoff, CHUNK)], idx_d)

                @pl.loop(0, CHUNK // 16)
                def _(j):
                    vec = idx_d[pl.ds(j * 16, 16)]
                    plsc.addupdate_scatter(cnt_v, [vec], ones16)

        pltpu.sync_copy(cnt_v, cnt_out.at[wid])

    return pl.kernel(
        body,
        out_type=jax.ShapeDtypeStruct((NW, NPAD), jnp.float32),
        mesh=_SC_MESH,
        scratch_types=[pltpu.VMEM((CHUNK,), jnp.int32),
                       pltpu.VMEM((NPAD,), jnp.float32)],
        compiler_params=_CP)


_seg_sum = _make_seg_sum()
_cnt_kernel = _make_cnt()


_BLK = 1000
_GRID = N // _BLK


def _dn():
    return (((1,), (1,)), ((), ()))


_PREC = lax.Precision.HIGHEST


def _cnt_recip_body(cnt_ref, out_ref):
    s = jnp.sum(cnt_ref[...], axis=0, keepdims=True)
    out_ref[...] = 1.0 / jnp.maximum(s, 1.0)


def _cnt_recip(cnt):
    return pl.pallas_call(
        _cnt_recip_body,
        out_shape=jax.ShapeDtypeStruct((1, NPAD), jnp.float32),
    )(cnt)


def _mm2_body(x_ref, wc_ref, wl_ref, b_ref, hc_ref, hl_ref):
    xx = x_ref[...]
    hc_ref[...] = lax.dot_general(xx, wc_ref[...], _dn(), precision=_PREC,
                                  preferred_element_type=jnp.float32)
    hl_ref[...] = lax.dot_general(xx, wl_ref[...], _dn(), precision=_PREC,
                                  preferred_element_type=jnp.float32) + b_ref[...]


def _mm2(x, wc, wl, b):
    return pl.pallas_call(
        _mm2_body,
        grid=(_GRID,),
        in_specs=[
            pl.BlockSpec((_BLK, D), lambda i: (i, 0)),
            pl.BlockSpec((H, D), lambda i: (0, 0)),
            pl.BlockSpec((H, D), lambda i: (0, 0)),
            pl.BlockSpec((1, H), lambda i: (0, 0)),
        ],
        out_specs=[
            pl.BlockSpec((_BLK, H), lambda i: (i, 0)),
            pl.BlockSpec((_BLK, H), lambda i: (i, 0)),
        ],
        out_shape=[
            jax.ShapeDtypeStruct((N, H), jnp.float32),
            jax.ShapeDtypeStruct((N, H), jnp.float32),
        ],
    )(x, wc, wl, b)


def _combine_mm2_body(hl_ref, a0_ref, a1_ref, cr_ref,
                      wc_ref, wl_ref, b_ref, hc_ref, hl2_ref):
    h1 = jnp.maximum(
        hl_ref[...] + (a0_ref[...] + a1_ref[...]) * cr_ref[...], 0.0)
    hc_ref[...] = lax.dot_general(h1, wc_ref[...], _dn(), precision=_PREC,
                                  preferred_element_type=jnp.float32)
    hl2_ref[...] = lax.dot_general(h1, wl_ref[...], _dn(), precision=_PREC,
                                   preferred_element_type=jnp.float32) + b_ref[...]


def _combine_mm2(hl, a0, a1, cr, wc, wl, b):
    return pl.pallas_call(
        _combine_mm2_body,
        grid=(_GRID,),
        in_specs=[
            pl.BlockSpec((_BLK, H), lambda i: (i, 0)),
            pl.BlockSpec((_BLK, H), lambda i: (i, 0)),
            pl.BlockSpec((_BLK, H), lambda i: (i, 0)),
            pl.BlockSpec((_BLK, 1), lambda i: (i, 0)),
            pl.BlockSpec((H, H), lambda i: (0, 0)),
            pl.BlockSpec((H, H), lambda i: (0, 0)),
            pl.BlockSpec((1, H), lambda i: (0, 0)),
        ],
        out_specs=[
            pl.BlockSpec((_BLK, H), lambda i: (i, 0)),
            pl.BlockSpec((_BLK, H), lambda i: (i, 0)),
        ],
        out_shape=[
            jax.ShapeDtypeStruct((N, H), jnp.float32),
            jax.ShapeDtypeStruct((N, H), jnp.float32),
        ],
    )(hl, a0, a1, cr, wc, wl, b)


def _final_body(hl_ref, a0_ref, a1_ref, cr_ref, out_ref):
    out_ref[...] = hl_ref[...] + (a0_ref[...] + a1_ref[...]) * cr_ref[...]


def _final(hl, a0, a1, cr):
    return pl.pallas_call(
        _final_body,
        grid=(_GRID,),
        in_specs=[
            pl.BlockSpec((_BLK, H), lambda i: (i, 0)),
            pl.BlockSpec((_BLK, H), lambda i: (i, 0)),
            pl.BlockSpec((_BLK, H), lambda i: (i, 0)),
            pl.BlockSpec((_BLK, 1), lambda i: (i, 0)),
        ],
        out_specs=pl.BlockSpec((_BLK, H), lambda i: (i, 0)),
        out_shape=jax.ShapeDtypeStruct((N, H), jnp.float32),
    )(hl, a0, a1, cr)


def kernel(x, edge_index, Wc0, bc0, Wl0, bl0, Wc1, bc1, Wl1, bl1):
    src = edge_index[0]
    dst = edge_index[1]
    b0 = (bl0 + bc0).reshape(1, H)
    b1 = (bl1 + bc1).reshape(1, H)

    hc0, hl0 = _mm2(x, Wc0, Wl0, b0)

    acc0, = _seg_sum(hc0, src, dst)
    cnt = _cnt_kernel(dst)
    crec = _cnt_recip(cnt).reshape(NPAD, 1)[:N]
    a0_0 = acc0[:N]
    a0_1 = acc0[NPAD:NPAD + N]

    hc1, hl1 = _combine_mm2(hl0, a0_0, a0_1, crec, Wc1, Wl1, b1)

    acc1, = _seg_sum(hc1, src, dst)
    a1_0 = acc1[:N]
    a1_1 = acc1[NPAD:NPAD + N]

    return _final(hl1, a1_0, a1_1, crec)


# speedup vs baseline: 9.6327x; 1.6754x over previous
import dataclasses
import functools

import jax
import jax.numpy as jnp
from jax import lax
from jax.experimental import pallas as pl
from jax.experimental.pallas import tpu as pltpu
from jax.experimental.pallas import tpu_sc as plsc

N = 10000
E = 320000
D = 128
H = 128
NPAD = 10240
NCORES = 2
NSUB = 16
NW = NCORES * NSUB
CHUNK = 128
NCHUNKS = E // CHUNK
ROWS_PER_SUB = NPAD // NSUB

_SC_MESH = plsc.VectorSubcoreMesh(core_axis_name="c", subcore_axis_name="s")

_CP = pltpu.CompilerParams()
if "needs_layout_passes" in pltpu.CompilerParams.__dataclass_fields__:
    _CP = dataclasses.replace(_CP, needs_layout_passes=False)


def _make_seg_sum():
    out_type = [jax.ShapeDtypeStruct((NCORES * NPAD, H), jnp.float32)]
    scratch = [
        pltpu.VMEM((2, CHUNK), jnp.int32),
        pltpu.VMEM((2, CHUNK), jnp.int32),
        pltpu.VMEM((CHUNK, H), jnp.float32),
        pltpu.VMEM((CHUNK, H), jnp.float32),
        pltpu.VMEM_SHARED((NPAD, H), jnp.float32),
        pltpu.SemaphoreType.DMA,
        pltpu.SemaphoreType.DMA,
    ]

    def body(hc_hbm, ei_hbm, acc_out, idx0, idx1, rows0, rows1,
             acc_sh, sem0, sem1):
        cid = lax.axis_index("c")
        sid = lax.axis_index("s")
        wid = cid * NSUB + sid
        base = sid * ROWS_PER_SUB
        zv = jnp.zeros((16,), jnp.float32)

        @pl.loop(0, CHUNK)
        def _(r):
            @pl.loop(0, H // 16)
            def _(c):
                rows0[r, pl.ds(c * 16, 16)] = zv

        @pl.loop(0, ROWS_PER_SUB // CHUNK)
        def _(k):
            off = pl.multiple_of(base + k * CHUNK, CHUNK)
            pltpu.sync_copy(rows0, acc_sh.at[pl.ds(off, CHUNK)])

        plsc.subcore_barrier()

        def load_idx(buf, c):
            eoff = pl.multiple_of(c * CHUNK, CHUNK)
            pltpu.sync_copy(ei_hbm.at[:, pl.ds(eoff, CHUNK)], buf)

        def start_gather(idx, rows, sem):
            pltpu.async_copy(hc_hbm.at[idx.at[0]], rows, sem)

        def wait_gather(idx, rows, sem):
            pltpu.make_async_copy(hc_hbm.at[idx.at[0]], rows, sem).wait()

        def scatter(idx, rows):
            pltpu.sync_copy(rows, acc_sh.at[idx.at[1]], add=True)

        max_iters = (NCHUNKS + NW - 1) // NW
        load_idx(idx0, wid)
        start_gather(idx0, rows0, sem0)
        load_idx(idx1, wid + NW)
        start_gather(idx1, rows1, sem1)

        @pl.loop(0, (max_iters + 1) // 2)
        def _(t):
            c0 = wid + (2 * t) * NW

            @pl.when(c0 < NCHUNKS)
            def _():
                wait_gather(idx0, rows0, sem0)
                scatter(idx0, rows0)
                c2 = c0 + 2 * NW

                @pl.when(c2 < NCHUNKS)
                def _():
                    load_idx(idx0, c2)
                    start_gather(idx0, rows0, sem0)

            c1 = c0 + NW

            @pl.when(c1 < NCHUNKS)
            def _():
                wait_gather(idx1, rows1, sem1)
                scatter(idx1, rows1)
                c3 = c1 + 2 * NW

                @pl.when(c3 < NCHUNKS)
                def _():
                    load_idx(idx1, c3)
                    start_gather(idx1, rows1, sem1)

        plsc.subcore_barrier()

        ooff = pl.multiple_of(cid * NPAD + base, CHUNK)
        pltpu.sync_copy(acc_sh.at[pl.ds(base, ROWS_PER_SUB)],
                        acc_out.at[pl.ds(ooff, ROWS_PER_SUB)])

    return pl.kernel(body, out_type=out_type, mesh=_SC_MESH,
                     scratch_types=scratch)


def _make_cnt():
    def body(dst_hbm, cnt_out, idx_d, cnt_v):
        cid = lax.axis_index("c")
        sid = lax.axis_index("s")
        wid = cid * NSUB + sid
        zv = jnp.zeros((16,), jnp.float32)
        ones16 = jnp.ones((16,), jnp.float32)

        @pl.loop(0, NPAD // 16)
        def _(r):
            cnt_v[pl.ds(r * 16, 16)] = zv

        max_iters = (NCHUNKS + NW - 1) // NW

        @pl.loop(0, max_iters)
        def _(i):
            c = wid + i * NW

            @pl.when(c < NCHUNKS)
            def _():
                eoff = pl.multiple_of(c * CHUNK, CHUNK)
                pltpu.sync_copy(dst_hbm.at[pl.ds(eoff, CHUNK)], idx_d)

                @pl.loop(0, CHUNK // 16)
                def _(j):
                    vec = idx_d[pl.ds(j * 16, 16)]
                    plsc.addupdate_scatter(cnt_v, [vec], ones16)

        pltpu.sync_copy(cnt_v, cnt_out.at[wid])

    return pl.kernel(
        body,
        out_type=jax.ShapeDtypeStruct((NW, NPAD), jnp.float32),
        mesh=_SC_MESH,
        scratch_types=[pltpu.VMEM((CHUNK,), jnp.int32),
                       pltpu.VMEM((NPAD,), jnp.float32)],
        compiler_params=_CP)


_seg_sum = _make_seg_sum()
_cnt_kernel = _make_cnt()


_BLK = 1000
_GRID = N // _BLK


def _dn():
    return (((1,), (1,)), ((), ()))


_PREC = lax.Precision.HIGHEST


def _cnt_recip_body(cnt_ref, out_ref):
    s = jnp.sum(cnt_ref[...], axis=0, keepdims=True)
    out_ref[...] = 1.0 / jnp.maximum(s, 1.0)


def _cnt_recip(cnt):
    return pl.pallas_call(
        _cnt_recip_body,
        out_shape=jax.ShapeDtypeStruct((1, NPAD), jnp.float32),
    )(cnt)


def _mm2_body(x_ref, wc_ref, wl_ref, b_ref, hc_ref, hl_ref):
    xx = x_ref[...]
    hc_ref[...] = lax.dot_general(xx, wc_ref[...], _dn(), precision=_PREC,
                                  preferred_element_type=jnp.float32)
    hl_ref[...] = lax.dot_general(xx, wl_ref[...], _dn(), precision=_PREC,
                                  preferred_element_type=jnp.float32) + b_ref[...]


def _mm2(x, wc, wl, b):
    return pl.pallas_call(
        _mm2_body,
        grid=(_GRID,),
        in_specs=[
            pl.BlockSpec((_BLK, D), lambda i: (i, 0)),
            pl.BlockSpec((H, D), lambda i: (0, 0)),
            pl.BlockSpec((H, D), lambda i: (0, 0)),
            pl.BlockSpec((1, H), lambda i: (0, 0)),
        ],
        out_specs=[
            pl.BlockSpec((_BLK, H), lambda i: (i, 0)),
            pl.BlockSpec((_BLK, H), lambda i: (i, 0)),
        ],
        out_shape=[
            jax.ShapeDtypeStruct((N, H), jnp.float32),
            jax.ShapeDtypeStruct((N, H), jnp.float32),
        ],
    )(x, wc, wl, b)


def _combine_mm2_body(hl_ref, a0_ref, a1_ref, cr_ref,
                      wc_ref, wl_ref, b_ref, hc_ref, hl2_ref):
    h1 = jnp.maximum(
        hl_ref[...] + (a0_ref[...] + a1_ref[...]) * cr_ref[...], 0.0)
    hc_ref[...] = lax.dot_general(h1, wc_ref[...], _dn(), precision=_PREC,
                                  preferred_element_type=jnp.float32)
    hl2_ref[...] = lax.dot_general(h1, wl_ref[...], _dn(), precision=_PREC,
                                   preferred_element_type=jnp.float32) + b_ref[...]


def _combine_mm2(hl, a0, a1, cr, wc, wl, b):
    return pl.pallas_call(
        _combine_mm2_body,
        grid=(_GRID,),
        in_specs=[
            pl.BlockSpec((_BLK, H), lambda i: (i, 0)),
            pl.BlockSpec((_BLK, H), lambda i: (i, 0)),
            pl.BlockSpec((_BLK, H), lambda i: (i, 0)),
            pl.BlockSpec((_BLK, 1), lambda i: (i, 0)),
            pl.BlockSpec((H, H), lambda i: (0, 0)),
            pl.BlockSpec((H, H), lambda i: (0, 0)),
            pl.BlockSpec((1, H), lambda i: (0, 0)),
        ],
        out_specs=[
            pl.BlockSpec((_BLK, H), lambda i: (i, 0)),
            pl.BlockSpec((_BLK, H), lambda i: (i, 0)),
        ],
        out_shape=[
            jax.ShapeDtypeStruct((N, H), jnp.float32),
            jax.ShapeDtypeStruct((N, H), jnp.float32),
        ],
    )(hl, a0, a1, cr, wc, wl, b)


def _final_body(hl_ref, a0_ref, a1_ref, cr_ref, out_ref):
    out_ref[...] = hl_ref[...] + (a0_ref[...] + a1_ref[...]) * cr_ref[...]


def _final(hl, a0, a1, cr):
    return pl.pallas_call(
        _final_body,
        grid=(_GRID,),
        in_specs=[
            pl.BlockSpec((_BLK, H), lambda i: (i, 0)),
            pl.BlockSpec((_BLK, H), lambda i: (i, 0)),
            pl.BlockSpec((_BLK, H), lambda i: (i, 0)),
            pl.BlockSpec((_BLK, 1), lambda i: (i, 0)),
        ],
        out_specs=pl.BlockSpec((_BLK, H), lambda i: (i, 0)),
        out_shape=jax.ShapeDtypeStruct((N, H), jnp.float32),
    )(hl, a0, a1, cr)


def kernel(x, edge_index, Wc0, bc0, Wl0, bl0, Wc1, bc1, Wl1, bl1):
    dst = edge_index[1]
    b0 = (bl0 + bc0).reshape(1, H)
    b1 = (bl1 + bc1).reshape(1, H)

    hc0, hl0 = _mm2(x, Wc0, Wl0, b0)

    acc0, = _seg_sum(hc0, edge_index)
    cnt = _cnt_kernel(dst)
    crec = _cnt_recip(cnt).reshape(NPAD, 1)[:N]
    a0_0 = acc0[:N]
    a0_1 = acc0[NPAD:NPAD + N]

    hc1, hl1 = _combine_mm2(hl0, a0_0, a0_1, crec, Wc1, Wl1, b1)

    acc1, = _seg_sum(hc1, edge_index)
    a1_0 = acc1[:N]
    a1_1 = acc1[NPAD:NPAD + N]

    return _final(hl1, a1_0, a1_1, crec)
